# baseline (speedup 1.0000x reference)
import functools

import numpy as np
import jax
import jax.numpy as jnp
from jax import lax
from jax.experimental import pallas as pl
from jax.experimental.pallas import tpu as pltpu
from jax.experimental.pallas import tpu_sc as plsc

_B, _L, _D = 2, 2048, 1024
_H, _DK = 16, 64
_N = _H * _DK
_F = _L // 2
_TOPK = 15


def _dft_consts():
    t = np.arange(_L)
    f = np.arange(_F)
    ang = 2.0 * np.pi * np.outer(f, t) / _L
    C = np.cos(ang)
    S = np.sin(ang)
    S[0, :] = (-1.0) ** t
    w = np.full((_F, 1), 2.0)
    w[0] = 1.0
    IC = (w / _L) * np.cos(ang)
    IS = -(w / _L) * np.sin(ang)
    IC[0, :] = 1.0 / _L
    IS[0, :] = ((-1.0) ** t) / _L
    CS = np.concatenate([C, S], axis=0)
    ICST = np.concatenate([IC, IS], axis=0).T
    return CS.astype(np.float32), ICST.astype(np.float32)


_CS_NP, _ICST_NP = _dft_consts()
_ICS_NP = np.ascontiguousarray(_ICST_NP.T)


_DN = (((1,), (0,)), ((), ()))


def _mm_kernel(x_ref, y_ref, b_ref, o_ref, *, precision):
    if precision == "split3":
        acc = _split3_dot(x_ref[0], y_ref[0])
    else:
        acc = lax.dot_general(x_ref[0], y_ref[0], _DN,
                              preferred_element_type=jnp.float32,
                              precision=precision)
    o_ref[0] = acc + b_ref[0:1, :]


def _bmm(x, y, bias, bm=512, bn=None, interpret=False,
         precision=lax.Precision.DEFAULT):
    Bx, M, K = x.shape
    By, K2, N = y.shape
    assert K == K2
    if bn is None:
        bn = N
    xmap = (lambda b, j, i: (b, i, 0)) if Bx > 1 else (lambda b, j, i: (0, i, 0))
    ymap = (lambda b, j, i: (b, 0, j)) if By > 1 else (lambda b, j, i: (0, 0, j))
    return pl.pallas_call(
        functools.partial(_mm_kernel, precision=precision),
        grid=(_B, N // bn, M // bm),
        in_specs=[
            pl.BlockSpec((1, bm, K), xmap),
            pl.BlockSpec((1, K, bn), ymap),
            pl.BlockSpec((8, bn), lambda b, j, i: (0, j)),
        ],
        out_specs=pl.BlockSpec((1, bm, bn), lambda b, j, i: (b, i, j)),
        out_shape=jax.ShapeDtypeStruct((_B, M, N), jnp.float32),
        interpret=interpret,
    )(x, y, bias)


def _pcross_kernel(a_ref, b_ref, o_ref):
    ar, ai = a_ref[0, :_F], a_ref[0, _F:]
    br, bi = b_ref[0, :_F], b_ref[0, _F:]
    rid = lax.broadcasted_iota(jnp.int32, (_F, _N), 0)
    is0 = rid == 0
    pr = ar * br + jnp.where(is0, 0.0, ai * bi)
    pi = jnp.where(is0, ai * bi, ar * bi - ai * br)
    o_ref[0, :_F] = pr
    o_ref[0, _F:] = pi


def _pcross(af, bf, interpret=False):
    return pl.pallas_call(
        _pcross_kernel,
        grid=(_B,),
        in_specs=[
            pl.BlockSpec((1, 2 * _F, _N), lambda b: (b, 0, 0)),
            pl.BlockSpec((1, 2 * _F, _N), lambda b: (b, 0, 0)),
        ],
        out_specs=pl.BlockSpec((1, 2 * _F, _N), lambda b: (b, 0, 0)),
        out_shape=jax.ShapeDtypeStruct((_B, 2 * _F, _N), jnp.float32),
        interpret=interpret,
    )(af, bf)


def _split3_dot(x, y):
    xh = x.astype(jnp.bfloat16)
    xl = (x - xh.astype(jnp.float32)).astype(jnp.bfloat16)
    yh = y.astype(jnp.bfloat16)
    yl = (y - yh.astype(jnp.float32)).astype(jnp.bfloat16)
    acc = lax.dot_general(xl, yh, _DN, preferred_element_type=jnp.float32)
    acc = acc + lax.dot_general(xh, yl, _DN, preferred_element_type=jnp.float32)
    return acc + lax.dot_general(xh, yh, _DN, preferred_element_type=jnp.float32)


def _proj_kernel(x_ref, w_ref, b_ref, o_ref):
    acc = lax.dot_general(x_ref[0], w_ref[0], _DN,
                          preferred_element_type=jnp.float32)
    o_ref[0] = acc + b_ref[0, 0:1, :]


def _proj_fused(x_all, w_all, b_all, bm=512, interpret=False):
    G = x_all.shape[0]
    return pl.pallas_call(
        _proj_kernel,
        grid=(G, _L // bm),
        in_specs=[
            pl.BlockSpec((1, bm, _D), lambda g, i: (g, i, 0)),
            pl.BlockSpec((1, _D, _N), lambda g, i: (g // _B, 0, 0)),
            pl.BlockSpec((1, 8, _N), lambda g, i: (g // _B, 0, 0)),
        ],
        out_specs=pl.BlockSpec((1, bm, _N), lambda g, i: (g, i, 0)),
        out_shape=jax.ShapeDtypeStruct((G, _L, _N), jnp.float32),
        interpret=interpret,
    )(x_all, w_all, b_all)


def _fwd_kernel(cs_ref, y_ref, o_ref):
    o_ref[0] = _split3_dot(cs_ref[0], y_ref[0])


def _fwd_fused(cs, y_all, bm=512, interpret=False):
    G = y_all.shape[0]
    return pl.pallas_call(
        _fwd_kernel,
        grid=(G, 2 * _F // bm),
        in_specs=[
            pl.BlockSpec((1, bm, _L), lambda g, i: (0, i, 0)),
            pl.BlockSpec((1, _L, _N), lambda g, i: (g, 0, 0)),
        ],
        out_specs=pl.BlockSpec((1, bm, _N), lambda g, i: (g, i, 0)),
        out_shape=jax.ShapeDtypeStruct((G, 2 * _F, _N), jnp.float32),
        interpret=interpret,
    )(cs, y_all)


def _cross(ar, ai, br, bi, n):
    rid = lax.broadcasted_iota(jnp.int32, (_F, n), 0)
    is0 = rid == 0
    pr = ar * br + jnp.where(is0, 0.0, ai * bi)
    pi = jnp.where(is0, ai * bi, ar * bi - ai * br)
    return pr, pi


def _corrt_kernel(yfq_ref, yfk_ref, ics_ref, o_ref):
    bn_ = yfq_ref.shape[2]
    ar, ai = yfq_ref[0, :_F], yfq_ref[0, _F:]
    br, bi = yfk_ref[0, :_F], yfk_ref[0, _F:]
    pr, pi = _cross(ar, ai, br, bi, bn_)
    pt = jnp.concatenate([pr, pi], axis=0).T
    o_ref[0] = _split3_dot(pt, ics_ref[0])


def _corrt_fused(yf, ics, bm=512, bn=1024, interpret=False):
    return pl.pallas_call(
        _corrt_kernel,
        grid=(_B, _L // bn, _N // bm),
        in_specs=[
            pl.BlockSpec((1, 2 * _F, bm), lambda b, j, i: (b, 0, i)),
            pl.BlockSpec((1, 2 * _F, bm), lambda b, j, i: (_B + b, 0, i)),
            pl.BlockSpec((1, 2 * _F, bn), lambda b, j, i: (0, 0, j)),
        ],
        out_specs=pl.BlockSpec((1, bm, bn), lambda b, j, i: (b, i, j)),
        out_shape=jax.ShapeDtypeStruct((_B, _N, _L), jnp.float32),
        interpret=interpret,
    )(yf, yf, ics)


def _wf_kernel(cs_ref, d_ref, w_ref, o_ref, wm_ref):
    @pl.when(pl.program_id(1) == 0)
    def _build():
        iota_d = lax.broadcasted_iota(jnp.int32, (_L, _N), 0)
        acc = jnp.zeros((_L, _N), jnp.float32)
        dly = d_ref[0]
        wts = w_ref[0]
        for i in range(_TOPK):
            acc = acc + jnp.where(iota_d == dly[i:i + 1, :],
                                  wts[i:i + 1, :], 0.0)
        wm_ref[...] = acc

    o_ref[0] = _split3_dot(cs_ref[0], wm_ref[...])


def _wf_fused(cs, dly_t, wts_t, bm=512, interpret=False):
    return pl.pallas_call(
        _wf_kernel,
        grid=(_B, 2 * _F // bm),
        in_specs=[
            pl.BlockSpec((1, bm, _L), lambda b, i: (0, i, 0)),
            pl.BlockSpec((1, 16, _N), lambda b, i: (b, 0, 0)),
            pl.BlockSpec((1, 16, _N), lambda b, i: (b, 0, 0)),
        ],
        out_specs=pl.BlockSpec((1, bm, _N), lambda b, i: (b, i, 0)),
        out_shape=jax.ShapeDtypeStruct((_B, 2 * _F, _N), jnp.float32),
        scratch_shapes=[pltpu.VMEM((_L, _N), jnp.float32)],
        interpret=interpret,
    )(cs, dly_t, wts_t)


def _agg_kernel(icst_ref, vf_ref, wf_ref, o_ref, p2_ref):
    @pl.when(pl.program_id(1) == 0)
    def _build():
        ar, ai = vf_ref[0, :_F], vf_ref[0, _F:]
        br, bi = wf_ref[0, :_F], wf_ref[0, _F:]
        pr, pi = _cross(ar, ai, br, bi, _N)
        p2_ref[:_F] = pr
        p2_ref[_F:] = pi

    o_ref[0] = _split3_dot(icst_ref[0], p2_ref[...])


def _agg_fused(icst, yf, wf, bm=256, interpret=False):
    return pl.pallas_call(
        _agg_kernel,
        grid=(_B, _L // bm),
        in_specs=[
            pl.BlockSpec((1, bm, 2 * _F), lambda b, i: (0, i, 0)),
            pl.BlockSpec((1, 2 * _F, _N), lambda b, i: (2 * _B + b, 0, 0)),
            pl.BlockSpec((1, 2 * _F, _N), lambda b, i: (b, 0, 0)),
        ],
        out_specs=pl.BlockSpec((1, bm, _N), lambda b, i: (b, i, 0)),
        out_shape=jax.ShapeDtypeStruct((_B, _L, _N), jnp.float32),
        scratch_shapes=[pltpu.VMEM((2 * _F, _N), jnp.float32)],
        interpret=interpret,
    )(icst, yf, wf)


def _pcross_t_kernel(a_ref, b_ref, o_ref):
    ar, ai = a_ref[0, :_F], a_ref[0, _F:]
    br, bi = b_ref[0, :_F], b_ref[0, _F:]
    rid = lax.broadcasted_iota(jnp.int32, (_F, _N), 0)
    is0 = rid == 0
    pr = ar * br + jnp.where(is0, 0.0, ai * bi)
    pi = jnp.where(is0, ai * bi, ar * bi - ai * br)
    o_ref[0, :, :_F] = pr.T
    o_ref[0, :, _F:] = pi.T


def _pcross_t(af, bf, interpret=False):
    return pl.pallas_call(
        _pcross_t_kernel,
        grid=(_B,),
        in_specs=[
            pl.BlockSpec((1, 2 * _F, _N), lambda b: (b, 0, 0)),
            pl.BlockSpec((1, 2 * _F, _N), lambda b: (b, 0, 0)),
        ],
        out_specs=pl.BlockSpec((1, _N, 2 * _F), lambda b: (b, 0, 0)),
        out_shape=jax.ShapeDtypeStruct((_B, _N, 2 * _F), jnp.float32),
        interpret=interpret,
    )(af, bf)


_R = _B * _N
_NW = 32
_RPW = _R // _NW


def _sc_topk(corrt):
    mesh = plsc.VectorSubcoreMesh(core_axis_name="c", subcore_axis_name="s")
    chunk = 16

    @functools.partial(
        pl.kernel,
        out_type=(jax.ShapeDtypeStruct((_R, 128), jnp.int32),
                  jax.ShapeDtypeStruct((_R, 128), jnp.float32)),
        mesh=mesh,
        scratch_types=[
            pltpu.VMEM((chunk, _L), jnp.float32),
            pltpu.VMEM((_RPW, 128), jnp.int32),
            pltpu.VMEM((_RPW, 128), jnp.float32),
        ],
        compiler_params=pltpu.CompilerParams(needs_layout_passes=False),
    )
    def k(corr_hbm, dly_hbm, wts_hbm, chunk_v, dly_v, wts_v):
        wid = lax.axis_index("s") * 2 + lax.axis_index("c")
        base = wid * _RPW
        lanes = lax.iota(jnp.int32, 16)
        neg = jnp.full((16,), -3.0e38, jnp.float32)
        zero16 = jnp.zeros((16,), jnp.int32)

        def do_chunk(ci, _):
            pltpu.sync_copy(corr_hbm.at[pl.ds(base + ci * chunk, chunk)],
                            chunk_v)

            def topk_pair(pr, _):
                def merge(j, carry):
                    cv0, ci0, cv1, ci1 = carry
                    ix = lanes + j * 16
                    x0 = chunk_v[2 * pr, pl.ds(j * 16, 16)]
                    x1 = chunk_v[2 * pr + 1, pl.ds(j * 16, 16)]
                    s0, si0 = plsc.sort_key_val(x0, ix)
                    s1, si1 = plsc.sort_key_val(x1, ix)
                    m0 = cv0 >= s0
                    m1 = cv1 >= s1
                    cv0, ci0 = plsc.sort_key_val(
                        jnp.where(m0, cv0, s0), jnp.where(m0, ci0, si0),
                        descending=True)
                    cv1, ci1 = plsc.sort_key_val(
                        jnp.where(m1, cv1, s1), jnp.where(m1, ci1, si1),
                        descending=True)
                    return cv0, ci0, cv1, ci1

                cv0, ci0, cv1, ci1 = lax.fori_loop(
                    0, _L // 16, merge, (neg, zero16, neg, zero16))

                def finish(cv, ci, row):
                    e = jnp.exp(cv - jnp.max(cv, axis=0))
                    e = jnp.where(lanes < _TOPK, e, 0.0)
                    w = e / jnp.sum(e, axis=0)
                    dly_v[row, pl.ds(0, 16)] = ci
                    wts_v[row, pl.ds(0, 16)] = w

                finish(cv0, ci0, ci * chunk + 2 * pr)
                finish(cv1, ci1, ci * chunk + 2 * pr + 1)
                return 0

            lax.fori_loop(0, chunk // 2, topk_pair, 0)
            return 0

        lax.fori_loop(0, _RPW // chunk, do_chunk, 0)
        pltpu.sync_copy(dly_v, dly_hbm.at[pl.ds(base, _RPW)])
        pltpu.sync_copy(wts_v, wts_hbm.at[pl.ds(base, _RPW)])

    return k(corrt)


def _wmat_kernel(d_ref, w_ref, o_ref):
    bd = o_ref.shape[1]
    dly = d_ref[0]
    wts = w_ref[0]
    iota_d = (lax.broadcasted_iota(jnp.int32, (bd, _N), 0)
              + pl.program_id(1) * bd)
    acc = jnp.zeros((bd, _N), jnp.float32)
    for i in range(_TOPK):
        acc = acc + jnp.where(iota_d == dly[i:i + 1, :], wts[i:i + 1, :], 0.0)
    o_ref[0] = acc


def _wmat_build(dly_t, wts_t, bd=512, interpret=False):
    return pl.pallas_call(
        _wmat_kernel,
        grid=(_B, _L // bd),
        in_specs=[
            pl.BlockSpec((1, 16, _N), lambda b, i: (b, 0, 0)),
            pl.BlockSpec((1, 16, _N), lambda b, i: (b, 0, 0)),
        ],
        out_specs=pl.BlockSpec((1, bd, _N), lambda b, i: (b, i, 0)),
        out_shape=jax.ShapeDtypeStruct((_B, _L, _N), jnp.float32),
        interpret=interpret,
    )(dly_t, wts_t)


def _topk_kernel(c_ref, o_ref):
    x = c_ref[0]
    iota_d = lax.broadcasted_iota(jnp.int32, (_L, _N), 0)
    neg = jnp.float32(-3.0e38)
    vals = []
    args = []
    for _ in range(_TOPK):
        m = jnp.max(x, axis=0, keepdims=True)
        am = jnp.min(jnp.where(x == m, iota_d, _L), axis=0, keepdims=True)
        x = jnp.where(iota_d == am, neg, x)
        vals.append(m)
        args.append(am)
    V = jnp.concatenate(vals, axis=0)
    e = jnp.exp(V - V[0:1, :])
    sm = e / jnp.sum(e, axis=0, keepdims=True)
    acc = jnp.zeros((_L, _N), jnp.float32)
    for i in range(_TOPK):
        acc = acc + jnp.where(iota_d == args[i], sm[i:i + 1, :], 0.0)
    o_ref[0] = acc


def _topk_weights(corr, interpret=False):
    return pl.pallas_call(
        _topk_kernel,
        grid=(_B,),
        in_specs=[pl.BlockSpec((1, _L, _N), lambda b: (b, 0, 0))],
        out_specs=pl.BlockSpec((1, _L, _N), lambda b: (b, 0, 0)),
        out_shape=jax.ShapeDtypeStruct((_B, _L, _N), jnp.float32),
        interpret=interpret,
    )(corr)


def _run(queries, keys, values, Wq, bq, Wk, bk, Wv, bv, Wo, bo,
         interpret=False):
    cs = jnp.asarray(_CS_NP)[None]
    icst = jnp.asarray(_ICST_NP)[None]
    ics = jnp.asarray(_ICS_NP)[None]
    bo8 = jnp.broadcast_to(bo, (8, _D))

    x_all = jnp.concatenate([queries, keys, values], axis=0)
    w_all = jnp.stack([Wq, Wk, Wv])
    b_all = jnp.stack([jnp.broadcast_to(bq, (8, _N)),
                       jnp.broadcast_to(bk, (8, _N)),
                       jnp.broadcast_to(bv, (8, _N))])
    y = _proj_fused(x_all, w_all, b_all, interpret=interpret)
    yf = _fwd_fused(cs, y, interpret=interpret)
    pt = _pcross_t(yf[:_B], yf[_B:2 * _B], interpret=interpret)
    zl = jnp.zeros((8, _L), jnp.float32)
    corrt = _bmm(pt, ics, zl, bn=1024, interpret=interpret,
                 precision="split3")
    dly, wts = _sc_topk(corrt.reshape(_R, _L))
    dly_t = dly[:, :16].reshape(_B, _N, 16).transpose(0, 2, 1)
    wts_t = wts[:, :16].reshape(_B, _N, 16).transpose(0, 2, 1)
    wf = _wf_fused(cs, dly_t, wts_t, interpret=interpret)
    agg = _agg_fused(icst, yf, wf, interpret=interpret)
    return _bmm(agg, Wo[None], bo8, interpret=interpret)


def kernel(queries, keys, values, Wq, bq, Wk, bk, Wv, bv, Wo, bo):
    return _run(queries, keys, values, Wq, bq, Wk, bk, Wv, bv, Wo, bo)


# speedup vs baseline: 1.1933x; 1.1933x over previous
import functools

import numpy as np
import jax
import jax.numpy as jnp
from jax import lax
from jax.experimental import pallas as pl
from jax.experimental.pallas import tpu as pltpu
from jax.experimental.pallas import tpu_sc as plsc

_B, _L, _D = 2, 2048, 1024
_H, _DK = 16, 64
_N = _H * _DK
_F = _L // 2
_TOPK = 15


def _dft_consts():
    t = np.arange(_L)
    f = np.arange(_F)
    ang = 2.0 * np.pi * np.outer(f, t) / _L
    C = np.cos(ang)
    S = np.sin(ang)
    S[0, :] = (-1.0) ** t
    w = np.full((_F, 1), 2.0)
    w[0] = 1.0
    IC = (w / _L) * np.cos(ang)
    IS = -(w / _L) * np.sin(ang)
    IC[0, :] = 1.0 / _L
    IS[0, :] = ((-1.0) ** t) / _L
    CS = np.concatenate([C, S], axis=0)
    ICST = np.concatenate([IC, IS], axis=0).T
    return CS.astype(np.float32), ICST.astype(np.float32)


_CS_NP, _ICST_NP = _dft_consts()
_ICS_NP = np.ascontiguousarray(_ICST_NP.T)


_DN = (((1,), (0,)), ((), ()))


def _mm_kernel(x_ref, y_ref, b_ref, o_ref, *, precision):
    if precision == "split3":
        acc = _split3_dot(x_ref[0], y_ref[0])
    else:
        acc = lax.dot_general(x_ref[0], y_ref[0], _DN,
                              preferred_element_type=jnp.float32,
                              precision=precision)
    o_ref[0] = acc + b_ref[0:1, :]


def _bmm(x, y, bias, bm=512, bn=None, interpret=False,
         precision=lax.Precision.DEFAULT):
    Bx, M, K = x.shape
    By, K2, N = y.shape
    assert K == K2
    if bn is None:
        bn = N
    xmap = (lambda b, j, i: (b, i, 0)) if Bx > 1 else (lambda b, j, i: (0, i, 0))
    ymap = (lambda b, j, i: (b, 0, j)) if By > 1 else (lambda b, j, i: (0, 0, j))
    return pl.pallas_call(
        functools.partial(_mm_kernel, precision=precision),
        grid=(_B, N // bn, M // bm),
        in_specs=[
            pl.BlockSpec((1, bm, K), xmap),
            pl.BlockSpec((1, K, bn), ymap),
            pl.BlockSpec((8, bn), lambda b, j, i: (0, j)),
        ],
        out_specs=pl.BlockSpec((1, bm, bn), lambda b, j, i: (b, i, j)),
        out_shape=jax.ShapeDtypeStruct((_B, M, N), jnp.float32),
        interpret=interpret,
    )(x, y, bias)


def _pcross_kernel(a_ref, b_ref, o_ref):
    ar, ai = a_ref[0, :_F], a_ref[0, _F:]
    br, bi = b_ref[0, :_F], b_ref[0, _F:]
    rid = lax.broadcasted_iota(jnp.int32, (_F, _N), 0)
    is0 = rid == 0
    pr = ar * br + jnp.where(is0, 0.0, ai * bi)
    pi = jnp.where(is0, ai * bi, ar * bi - ai * br)
    o_ref[0, :_F] = pr
    o_ref[0, _F:] = pi


def _pcross(af, bf, interpret=False):
    return pl.pallas_call(
        _pcross_kernel,
        grid=(_B,),
        in_specs=[
            pl.BlockSpec((1, 2 * _F, _N), lambda b: (b, 0, 0)),
            pl.BlockSpec((1, 2 * _F, _N), lambda b: (b, 0, 0)),
        ],
        out_specs=pl.BlockSpec((1, 2 * _F, _N), lambda b: (b, 0, 0)),
        out_shape=jax.ShapeDtypeStruct((_B, 2 * _F, _N), jnp.float32),
        interpret=interpret,
    )(af, bf)


def _split3_dot(x, y):
    xh = x.astype(jnp.bfloat16)
    xl = (x - xh.astype(jnp.float32)).astype(jnp.bfloat16)
    yh = y.astype(jnp.bfloat16)
    yl = (y - yh.astype(jnp.float32)).astype(jnp.bfloat16)
    acc = lax.dot_general(xl, yh, _DN, preferred_element_type=jnp.float32)
    acc = acc + lax.dot_general(xh, yl, _DN, preferred_element_type=jnp.float32)
    return acc + lax.dot_general(xh, yh, _DN, preferred_element_type=jnp.float32)


def _proj_kernel(x_ref, w_ref, b_ref, o_ref):
    acc = lax.dot_general(x_ref[0], w_ref[0], _DN,
                          preferred_element_type=jnp.float32)
    o_ref[0] = acc + b_ref[0, 0:1, :]


def _proj_fused(x_all, w_all, b_all, bm=512, interpret=False):
    G = x_all.shape[0]
    return pl.pallas_call(
        _proj_kernel,
        grid=(G, _L // bm),
        in_specs=[
            pl.BlockSpec((1, bm, _D), lambda g, i: (g, i, 0)),
            pl.BlockSpec((1, _D, _N), lambda g, i: (g // _B, 0, 0)),
            pl.BlockSpec((1, 8, _N), lambda g, i: (g // _B, 0, 0)),
        ],
        out_specs=pl.BlockSpec((1, bm, _N), lambda g, i: (g, i, 0)),
        out_shape=jax.ShapeDtypeStruct((G, _L, _N), jnp.float32),
        interpret=interpret,
    )(x_all, w_all, b_all)


def _fwd_kernel(cs_ref, y_ref, o_ref):
    o_ref[0] = _split3_dot(cs_ref[0], y_ref[0])


def _fwd_fused(cs, y_all, bm=512, interpret=False):
    G = y_all.shape[0]
    return pl.pallas_call(
        _fwd_kernel,
        grid=(G, 2 * _F // bm),
        in_specs=[
            pl.BlockSpec((1, bm, _L), lambda g, i: (0, i, 0)),
            pl.BlockSpec((1, _L, _N), lambda g, i: (g, 0, 0)),
        ],
        out_specs=pl.BlockSpec((1, bm, _N), lambda g, i: (g, i, 0)),
        out_shape=jax.ShapeDtypeStruct((G, 2 * _F, _N), jnp.float32),
        interpret=interpret,
    )(cs, y_all)


def _cross(ar, ai, br, bi, n):
    rid = lax.broadcasted_iota(jnp.int32, (_F, n), 0)
    is0 = rid == 0
    pr = ar * br + jnp.where(is0, 0.0, ai * bi)
    pi = jnp.where(is0, ai * bi, ar * bi - ai * br)
    return pr, pi


def _corrt_kernel(yfq_ref, yfk_ref, ics_ref, o_ref):
    bn_ = yfq_ref.shape[2]
    ar, ai = yfq_ref[0, :_F], yfq_ref[0, _F:]
    br, bi = yfk_ref[0, :_F], yfk_ref[0, _F:]
    pr, pi = _cross(ar, ai, br, bi, bn_)
    pt = jnp.concatenate([pr, pi], axis=0).T
    o_ref[0] = _split3_dot(pt, ics_ref[0])


def _corrt_fused(qf, kf, ics, bm=512, bn=1024, interpret=False):
    return pl.pallas_call(
        _corrt_kernel,
        grid=(_B, _L // bn, _N // bm),
        in_specs=[
            pl.BlockSpec((1, 2 * _F, bm), lambda b, j, i: (b, 0, i)),
            pl.BlockSpec((1, 2 * _F, bm), lambda b, j, i: (b, 0, i)),
            pl.BlockSpec((1, 2 * _F, bn), lambda b, j, i: (0, 0, j)),
        ],
        out_specs=pl.BlockSpec((1, bm, bn), lambda b, j, i: (b, i, j)),
        out_shape=jax.ShapeDtypeStruct((_B, _N, _L), jnp.float32),
        interpret=interpret,
    )(qf, kf, ics)


def _wf_kernel(cs_ref, d_ref, w_ref, o_ref, wm_ref):
    @pl.when(pl.program_id(1) == 0)
    def _build():
        iota_d = lax.broadcasted_iota(jnp.int32, (_L, _N), 0)
        acc = jnp.zeros((_L, _N), jnp.float32)
        dly = d_ref[0]
        wts = w_ref[0]
        for i in range(_TOPK):
            acc = acc + jnp.where(iota_d == dly[i:i + 1, :],
                                  wts[i:i + 1, :], 0.0)
        wm_ref[...] = acc

    o_ref[0] = _split3_dot(cs_ref[0], wm_ref[...])


def _wf_fused(cs, dly_t, wts_t, bm=512, interpret=False):
    return pl.pallas_call(
        _wf_kernel,
        grid=(_B, 2 * _F // bm),
        in_specs=[
            pl.BlockSpec((1, bm, _L), lambda b, i: (0, i, 0)),
            pl.BlockSpec((1, 16, _N), lambda b, i: (b, 0, 0)),
            pl.BlockSpec((1, 16, _N), lambda b, i: (b, 0, 0)),
        ],
        out_specs=pl.BlockSpec((1, bm, _N), lambda b, i: (b, i, 0)),
        out_shape=jax.ShapeDtypeStruct((_B, 2 * _F, _N), jnp.float32),
        scratch_shapes=[pltpu.VMEM((_L, _N), jnp.float32)],
        interpret=interpret,
    )(cs, dly_t, wts_t)


def _agg_kernel(icst_ref, vf_ref, wf_ref, o_ref, p2_ref):
    @pl.when(pl.program_id(1) == 0)
    def _build():
        ar, ai = vf_ref[0, :_F], vf_ref[0, _F:]
        br, bi = wf_ref[0, :_F], wf_ref[0, _F:]
        pr, pi = _cross(ar, ai, br, bi, _N)
        p2_ref[:_F] = pr
        p2_ref[_F:] = pi

    o_ref[0] = _split3_dot(icst_ref[0], p2_ref[...])


def _agg_fused(icst, yf, wf, bm=256, interpret=False):
    return pl.pallas_call(
        _agg_kernel,
        grid=(_B, _L // bm),
        in_specs=[
            pl.BlockSpec((1, bm, 2 * _F), lambda b, i: (0, i, 0)),
            pl.BlockSpec((1, 2 * _F, _N), lambda b, i: (2 * _B + b, 0, 0)),
            pl.BlockSpec((1, 2 * _F, _N), lambda b, i: (b, 0, 0)),
        ],
        out_specs=pl.BlockSpec((1, bm, _N), lambda b, i: (b, i, 0)),
        out_shape=jax.ShapeDtypeStruct((_B, _L, _N), jnp.float32),
        scratch_shapes=[pltpu.VMEM((2 * _F, _N), jnp.float32)],
        interpret=interpret,
    )(icst, yf, wf)


def _pcross_t_kernel(a_ref, b_ref, o_ref):
    ar, ai = a_ref[0, :_F], a_ref[0, _F:]
    br, bi = b_ref[0, :_F], b_ref[0, _F:]
    rid = lax.broadcasted_iota(jnp.int32, (_F, _N), 0)
    is0 = rid == 0
    pr = ar * br + jnp.where(is0, 0.0, ai * bi)
    pi = jnp.where(is0, ai * bi, ar * bi - ai * br)
    o_ref[0, :, :_F] = pr.T
    o_ref[0, :, _F:] = pi.T


def _pcross_t(af, bf, interpret=False):
    return pl.pallas_call(
        _pcross_t_kernel,
        grid=(_B,),
        in_specs=[
            pl.BlockSpec((1, 2 * _F, _N), lambda b: (b, 0, 0)),
            pl.BlockSpec((1, 2 * _F, _N), lambda b: (b, 0, 0)),
        ],
        out_specs=pl.BlockSpec((1, _N, 2 * _F), lambda b: (b, 0, 0)),
        out_shape=jax.ShapeDtypeStruct((_B, _N, 2 * _F), jnp.float32),
        interpret=interpret,
    )(af, bf)


_R = _B * _N
_NW = 32
_RPW = _R // _NW


def _sc_topk(corrt):
    mesh = plsc.VectorSubcoreMesh(core_axis_name="c", subcore_axis_name="s")
    chunk = 16

    @functools.partial(
        pl.kernel,
        out_type=(jax.ShapeDtypeStruct((_R, 128), jnp.int32),
                  jax.ShapeDtypeStruct((_R, 128), jnp.float32)),
        mesh=mesh,
        scratch_types=[
            pltpu.VMEM((chunk, _L), jnp.float32),
            pltpu.VMEM((_RPW, 128), jnp.int32),
            pltpu.VMEM((_RPW, 128), jnp.float32),
        ],
        compiler_params=pltpu.CompilerParams(needs_layout_passes=False),
    )
    def k(corr_hbm, dly_hbm, wts_hbm, chunk_v, dly_v, wts_v):
        wid = lax.axis_index("s") * 2 + lax.axis_index("c")
        base = wid * _RPW
        lanes = lax.iota(jnp.int32, 16)
        neg = jnp.full((16,), -3.0e38, jnp.float32)
        zero16 = jnp.zeros((16,), jnp.int32)

        def do_chunk(ci, _):
            pltpu.sync_copy(corr_hbm.at[pl.ds(base + ci * chunk, chunk)],
                            chunk_v)

            def topk_pair(pr, _):
                def merge(j, carry):
                    cv0, ci0, cv1, ci1 = carry
                    ix = lanes + j * 16
                    x0 = chunk_v[2 * pr, pl.ds(j * 16, 16)]
                    x1 = chunk_v[2 * pr + 1, pl.ds(j * 16, 16)]
                    s0, si0 = plsc.sort_key_val(x0, ix)
                    s1, si1 = plsc.sort_key_val(x1, ix)
                    m0 = cv0 >= s0
                    m1 = cv1 >= s1
                    cv0, ci0 = plsc.sort_key_val(
                        jnp.where(m0, cv0, s0), jnp.where(m0, ci0, si0),
                        descending=True)
                    cv1, ci1 = plsc.sort_key_val(
                        jnp.where(m1, cv1, s1), jnp.where(m1, ci1, si1),
                        descending=True)
                    return cv0, ci0, cv1, ci1

                cv0, ci0, cv1, ci1 = lax.fori_loop(
                    0, _L // 16, merge, (neg, zero16, neg, zero16))

                def finish(cv, ci, row):
                    e = jnp.exp(cv - jnp.max(cv, axis=0))
                    e = jnp.where(lanes < _TOPK, e, 0.0)
                    w = e / jnp.sum(e, axis=0)
                    dly_v[row, pl.ds(0, 16)] = ci
                    wts_v[row, pl.ds(0, 16)] = w

                finish(cv0, ci0, ci * chunk + 2 * pr)
                finish(cv1, ci1, ci * chunk + 2 * pr + 1)
                return 0

            lax.fori_loop(0, chunk // 2, topk_pair, 0)
            return 0

        lax.fori_loop(0, _RPW // chunk, do_chunk, 0)
        pltpu.sync_copy(dly_v, dly_hbm.at[pl.ds(base, _RPW)])
        pltpu.sync_copy(wts_v, wts_hbm.at[pl.ds(base, _RPW)])

    return k(corrt)


def _wmat_kernel(d_ref, w_ref, o_ref):
    bd = o_ref.shape[1]
    dly = d_ref[0]
    wts = w_ref[0]
    iota_d = (lax.broadcasted_iota(jnp.int32, (bd, _N), 0)
              + pl.program_id(1) * bd)
    acc = jnp.zeros((bd, _N), jnp.float32)
    for i in range(_TOPK):
        acc = acc + jnp.where(iota_d == dly[i:i + 1, :], wts[i:i + 1, :], 0.0)
    o_ref[0] = acc


def _wmat_build(dly_t, wts_t, bd=512, interpret=False):
    return pl.pallas_call(
        _wmat_kernel,
        grid=(_B, _L // bd),
        in_specs=[
            pl.BlockSpec((1, 16, _N), lambda b, i: (b, 0, 0)),
            pl.BlockSpec((1, 16, _N), lambda b, i: (b, 0, 0)),
        ],
        out_specs=pl.BlockSpec((1, bd, _N), lambda b, i: (b, i, 0)),
        out_shape=jax.ShapeDtypeStruct((_B, _L, _N), jnp.float32),
        interpret=interpret,
    )(dly_t, wts_t)


def _topk_kernel(c_ref, o_ref):
    x = c_ref[0]
    iota_d = lax.broadcasted_iota(jnp.int32, (_L, _N), 0)
    neg = jnp.float32(-3.0e38)
    vals = []
    args = []
    for _ in range(_TOPK):
        m = jnp.max(x, axis=0, keepdims=True)
        am = jnp.min(jnp.where(x == m, iota_d, _L), axis=0, keepdims=True)
        x = jnp.where(iota_d == am, neg, x)
        vals.append(m)
        args.append(am)
    V = jnp.concatenate(vals, axis=0)
    e = jnp.exp(V - V[0:1, :])
    sm = e / jnp.sum(e, axis=0, keepdims=True)
    acc = jnp.zeros((_L, _N), jnp.float32)
    for i in range(_TOPK):
        acc = acc + jnp.where(iota_d == args[i], sm[i:i + 1, :], 0.0)
    o_ref[0] = acc


def _topk_weights(corr, interpret=False):
    return pl.pallas_call(
        _topk_kernel,
        grid=(_B,),
        in_specs=[pl.BlockSpec((1, _L, _N), lambda b: (b, 0, 0))],
        out_specs=pl.BlockSpec((1, _L, _N), lambda b: (b, 0, 0)),
        out_shape=jax.ShapeDtypeStruct((_B, _L, _N), jnp.float32),
        interpret=interpret,
    )(corr)


def _run(queries, keys, values, Wq, bq, Wk, bk, Wv, bv, Wo, bo,
         interpret=False):
    cs = jnp.asarray(_CS_NP)[None]
    icst = jnp.asarray(_ICST_NP)[None]
    ics = jnp.asarray(_ICS_NP)[None]
    bo8 = jnp.broadcast_to(bo, (8, _D))

    hi = "split3"
    z = jnp.zeros((8, _N), jnp.float32)
    bq8 = jnp.broadcast_to(bq, (8, _N))
    bk8 = jnp.broadcast_to(bk, (8, _N))
    bv8 = jnp.broadcast_to(bv, (8, _N))
    q = _bmm(queries, Wq[None], bq8, interpret=interpret)
    k = _bmm(keys, Wk[None], bk8, interpret=interpret)
    v = _bmm(values, Wv[None], bv8, interpret=interpret)
    qf = _bmm(cs, q, z, interpret=interpret, precision=hi)
    kf = _bmm(cs, k, z, interpret=interpret, precision=hi)
    vf = _bmm(cs, v, z, interpret=interpret, precision=hi)
    corrt = _corrt_fused(qf, kf, ics, interpret=interpret)
    dly, wts = _sc_topk(corrt.reshape(_R, _L))
    dly_t = dly[:, :16].reshape(_B, _N, 16).transpose(0, 2, 1)
    wts_t = wts[:, :16].reshape(_B, _N, 16).transpose(0, 2, 1)
    wmat = _wmat_build(dly_t, wts_t, interpret=interpret)
    wf = _bmm(cs, wmat, z, interpret=interpret, precision=hi)
    p2 = _pcross(vf, wf, interpret=interpret)
    agg = _bmm(icst, p2, z, interpret=interpret, precision=hi)
    return _bmm(agg, Wo[None], bo8, interpret=interpret)


def kernel(queries, keys, values, Wq, bq, Wk, bk, Wv, bv, Wo, bo):
    return _run(queries, keys, values, Wq, bq, Wk, bk, Wv, bv, Wo, bo)


# speedup vs baseline: 1.1990x; 1.0048x over previous
import functools

import numpy as np
import jax
import jax.numpy as jnp
from jax import lax
from jax.experimental import pallas as pl
from jax.experimental.pallas import tpu as pltpu
from jax.experimental.pallas import tpu_sc as plsc

_B, _L, _D = 2, 2048, 1024
_H, _DK = 16, 64
_N = _H * _DK
_F = _L // 2
_TOPK = 15


def _dft_consts():
    t = np.arange(_L)
    f = np.arange(_F)
    ang = 2.0 * np.pi * np.outer(f, t) / _L
    C = np.cos(ang)
    S = np.sin(ang)
    S[0, :] = (-1.0) ** t
    w = np.full((_F, 1), 2.0)
    w[0] = 1.0
    IC = (w / _L) * np.cos(ang)
    IS = -(w / _L) * np.sin(ang)
    IC[0, :] = 1.0 / _L
    IS[0, :] = ((-1.0) ** t) / _L
    CS = np.concatenate([C, S], axis=0)
    ICST = np.concatenate([IC, IS], axis=0).T
    return CS.astype(np.float32), ICST.astype(np.float32)


_CS_NP, _ICST_NP = _dft_consts()
_ICS_NP = np.ascontiguousarray(_ICST_NP.T)

import ml_dtypes as _mld


def _np_bf16_split(a):
    h = a.astype(_mld.bfloat16)
    l = (a - h.astype(np.float32)).astype(_mld.bfloat16)
    return h, l


_CS_H, _CS_L = _np_bf16_split(_CS_NP)
_ICST_H, _ICST_L = _np_bf16_split(_ICST_NP)
_ICS_H, _ICS_L = _np_bf16_split(_ICS_NP)


_DN = (((1,), (0,)), ((), ()))


def _mm_kernel(x_ref, y_ref, b_ref, o_ref, *, precision):
    if precision == "split3":
        acc = _split3_dot(x_ref[0], y_ref[0])
    else:
        acc = lax.dot_general(x_ref[0], y_ref[0], _DN,
                              preferred_element_type=jnp.float32,
                              precision=precision)
    o_ref[0] = acc + b_ref[0:1, :]


def _bmm(x, y, bias, bm=512, bn=None, interpret=False,
         precision=lax.Precision.DEFAULT):
    Bx, M, K = x.shape
    By, K2, N = y.shape
    assert K == K2
    if bn is None:
        bn = N
    xmap = (lambda b, j, i: (b, i, 0)) if Bx > 1 else (lambda b, j, i: (0, i, 0))
    ymap = (lambda b, j, i: (b, 0, j)) if By > 1 else (lambda b, j, i: (0, 0, j))
    return pl.pallas_call(
        functools.partial(_mm_kernel, precision=precision),
        grid=(_B, N // bn, M // bm),
        in_specs=[
            pl.BlockSpec((1, bm, K), xmap),
            pl.BlockSpec((1, K, bn), ymap),
            pl.BlockSpec((8, bn), lambda b, j, i: (0, j)),
        ],
        out_specs=pl.BlockSpec((1, bm, bn), lambda b, j, i: (b, i, j)),
        out_shape=jax.ShapeDtypeStruct((_B, M, N), jnp.float32),
        interpret=interpret,
    )(x, y, bias)


def _pcross_kernel(a_ref, b_ref, o_ref):
    ar, ai = a_ref[0, :_F], a_ref[0, _F:]
    br, bi = b_ref[0, :_F], b_ref[0, _F:]
    rid = lax.broadcasted_iota(jnp.int32, (_F, _N), 0)
    is0 = rid == 0
    pr = ar * br + jnp.where(is0, 0.0, ai * bi)
    pi = jnp.where(is0, ai * bi, ar * bi - ai * br)
    o_ref[0, :_F] = pr
    o_ref[0, _F:] = pi


def _pcross(af, bf, interpret=False):
    return pl.pallas_call(
        _pcross_kernel,
        grid=(_B,),
        in_specs=[
            pl.BlockSpec((1, 2 * _F, _N), lambda b: (b, 0, 0)),
            pl.BlockSpec((1, 2 * _F, _N), lambda b: (b, 0, 0)),
        ],
        out_specs=pl.BlockSpec((1, 2 * _F, _N), lambda b: (b, 0, 0)),
        out_shape=jax.ShapeDtypeStruct((_B, 2 * _F, _N), jnp.float32),
        interpret=interpret,
    )(af, bf)


def _split3_dot(x, y):
    xh = x.astype(jnp.bfloat16)
    xl = (x - xh.astype(jnp.float32)).astype(jnp.bfloat16)
    yh = y.astype(jnp.bfloat16)
    yl = (y - yh.astype(jnp.float32)).astype(jnp.bfloat16)
    acc = lax.dot_general(xl, yh, _DN, preferred_element_type=jnp.float32)
    acc = acc + lax.dot_general(xh, yl, _DN, preferred_element_type=jnp.float32)
    return acc + lax.dot_general(xh, yh, _DN, preferred_element_type=jnp.float32)


def _mm_psl_kernel(xh_ref, xl_ref, y_ref, o_ref):
    y = y_ref[0]
    yh = y.astype(jnp.bfloat16)
    yl = (y - yh.astype(jnp.float32)).astype(jnp.bfloat16)
    xh, xl = xh_ref[0], xl_ref[0]
    acc = lax.dot_general(xl, yh, _DN, preferred_element_type=jnp.float32)
    acc = acc + lax.dot_general(xh, yl, _DN, preferred_element_type=jnp.float32)
    o_ref[0] = acc + lax.dot_general(xh, yh, _DN,
                                     preferred_element_type=jnp.float32)


def _bmm_psl(xh, xl, y, bm=512, interpret=False):
    _, M, K = xh.shape
    _, K2, N = y.shape
    return pl.pallas_call(
        _mm_psl_kernel,
        grid=(_B, M // bm),
        in_specs=[
            pl.BlockSpec((1, bm, K), lambda b, i: (0, i, 0)),
            pl.BlockSpec((1, bm, K), lambda b, i: (0, i, 0)),
            pl.BlockSpec((1, K, N), lambda b, i: (b, 0, 0)),
        ],
        out_specs=pl.BlockSpec((1, bm, N), lambda b, i: (b, i, 0)),
        out_shape=jax.ShapeDtypeStruct((_B, M, N), jnp.float32),
        interpret=interpret,
    )(xh, xl, y)


def _proj_kernel(x_ref, w_ref, b_ref, o_ref):
    acc = lax.dot_general(x_ref[0], w_ref[0], _DN,
                          preferred_element_type=jnp.float32)
    o_ref[0] = acc + b_ref[0, 0:1, :]


def _proj_fused(x_all, w_all, b_all, bm=512, interpret=False):
    G = x_all.shape[0]
    return pl.pallas_call(
        _proj_kernel,
        grid=(G, _L // bm),
        in_specs=[
            pl.BlockSpec((1, bm, _D), lambda g, i: (g, i, 0)),
            pl.BlockSpec((1, _D, _N), lambda g, i: (g // _B, 0, 0)),
            pl.BlockSpec((1, 8, _N), lambda g, i: (g // _B, 0, 0)),
        ],
        out_specs=pl.BlockSpec((1, bm, _N), lambda g, i: (g, i, 0)),
        out_shape=jax.ShapeDtypeStruct((G, _L, _N), jnp.float32),
        interpret=interpret,
    )(x_all, w_all, b_all)


def _fwd_kernel(cs_ref, y_ref, o_ref):
    o_ref[0] = _split3_dot(cs_ref[0], y_ref[0])


def _fwd_fused(cs, y_all, bm=512, interpret=False):
    G = y_all.shape[0]
    return pl.pallas_call(
        _fwd_kernel,
        grid=(G, 2 * _F // bm),
        in_specs=[
            pl.BlockSpec((1, bm, _L), lambda g, i: (0, i, 0)),
            pl.BlockSpec((1, _L, _N), lambda g, i: (g, 0, 0)),
        ],
        out_specs=pl.BlockSpec((1, bm, _N), lambda g, i: (g, i, 0)),
        out_shape=jax.ShapeDtypeStruct((G, 2 * _F, _N), jnp.float32),
        interpret=interpret,
    )(cs, y_all)


def _cross(ar, ai, br, bi, n):
    rid = lax.broadcasted_iota(jnp.int32, (_F, n), 0)
    is0 = rid == 0
    pr = ar * br + jnp.where(is0, 0.0, ai * bi)
    pi = jnp.where(is0, ai * bi, ar * bi - ai * br)
    return pr, pi


def _corrt_kernel(yfq_ref, yfk_ref, ih_ref, il_ref, o_ref):
    bn_ = yfq_ref.shape[2]
    ar, ai = yfq_ref[0, :_F], yfq_ref[0, _F:]
    br, bi = yfk_ref[0, :_F], yfk_ref[0, _F:]
    pr, pi = _cross(ar, ai, br, bi, bn_)
    pt = jnp.concatenate([pr, pi], axis=0).T
    pth = pt.astype(jnp.bfloat16)
    ptl = (pt - pth.astype(jnp.float32)).astype(jnp.bfloat16)
    ih, il = ih_ref[0], il_ref[0]
    acc = lax.dot_general(ptl, ih, _DN, preferred_element_type=jnp.float32)
    acc = acc + lax.dot_general(pth, il, _DN, preferred_element_type=jnp.float32)
    o_ref[0] = acc + lax.dot_general(pth, ih, _DN,
                                     preferred_element_type=jnp.float32)


def _corrt_fused(qf, kf, ih, il, bm=512, bn=1024, interpret=False):
    return pl.pallas_call(
        _corrt_kernel,
        grid=(_B, _L // bn, _N // bm),
        in_specs=[
            pl.BlockSpec((1, 2 * _F, bm), lambda b, j, i: (b, 0, i)),
            pl.BlockSpec((1, 2 * _F, bm), lambda b, j, i: (b, 0, i)),
            pl.BlockSpec((1, 2 * _F, bn), lambda b, j, i: (0, 0, j)),
            pl.BlockSpec((1, 2 * _F, bn), lambda b, j, i: (0, 0, j)),
        ],
        out_specs=pl.BlockSpec((1, bm, bn), lambda b, j, i: (b, i, j)),
        out_shape=jax.ShapeDtypeStruct((_B, _N, _L), jnp.float32),
        interpret=interpret,
    )(qf, kf, ih, il)


def _wf_kernel(cs_ref, d_ref, w_ref, o_ref, wm_ref):
    @pl.when(pl.program_id(1) == 0)
    def _build():
        iota_d = lax.broadcasted_iota(jnp.int32, (_L, _N), 0)
        acc = jnp.zeros((_L, _N), jnp.float32)
        dly = d_ref[0]
        wts = w_ref[0]
        for i in range(_TOPK):
            acc = acc + jnp.where(iota_d == dly[i:i + 1, :],
                                  wts[i:i + 1, :], 0.0)
        wm_ref[...] = acc

    o_ref[0] = _split3_dot(cs_ref[0], wm_ref[...])


def _wf_fused(cs, dly_t, wts_t, bm=512, interpret=False):
    return pl.pallas_call(
        _wf_kernel,
        grid=(_B, 2 * _F // bm),
        in_specs=[
            pl.BlockSpec((1, bm, _L), lambda b, i: (0, i, 0)),
            pl.BlockSpec((1, 16, _N), lambda b, i: (b, 0, 0)),
            pl.BlockSpec((1, 16, _N), lambda b, i: (b, 0, 0)),
        ],
        out_specs=pl.BlockSpec((1, bm, _N), lambda b, i: (b, i, 0)),
        out_shape=jax.ShapeDtypeStruct((_B, 2 * _F, _N), jnp.float32),
        scratch_shapes=[pltpu.VMEM((_L, _N), jnp.float32)],
        interpret=interpret,
    )(cs, dly_t, wts_t)


def _agg_kernel(icst_ref, vf_ref, wf_ref, o_ref, p2_ref):
    @pl.when(pl.program_id(1) == 0)
    def _build():
        ar, ai = vf_ref[0, :_F], vf_ref[0, _F:]
        br, bi = wf_ref[0, :_F], wf_ref[0, _F:]
        pr, pi = _cross(ar, ai, br, bi, _N)
        p2_ref[:_F] = pr
        p2_ref[_F:] = pi

    o_ref[0] = _split3_dot(icst_ref[0], p2_ref[...])


def _agg_fused(icst, yf, wf, bm=256, interpret=False):
    return pl.pallas_call(
        _agg_kernel,
        grid=(_B, _L // bm),
        in_specs=[
            pl.BlockSpec((1, bm, 2 * _F), lambda b, i: (0, i, 0)),
            pl.BlockSpec((1, 2 * _F, _N), lambda b, i: (2 * _B + b, 0, 0)),
            pl.BlockSpec((1, 2 * _F, _N), lambda b, i: (b, 0, 0)),
        ],
        out_specs=pl.BlockSpec((1, bm, _N), lambda b, i: (b, i, 0)),
        out_shape=jax.ShapeDtypeStruct((_B, _L, _N), jnp.float32),
        scratch_shapes=[pltpu.VMEM((2 * _F, _N), jnp.float32)],
        interpret=interpret,
    )(icst, yf, wf)


def _pcross_t_kernel(a_ref, b_ref, o_ref):
    ar, ai = a_ref[0, :_F], a_ref[0, _F:]
    br, bi = b_ref[0, :_F], b_ref[0, _F:]
    rid = lax.broadcasted_iota(jnp.int32, (_F, _N), 0)
    is0 = rid == 0
    pr = ar * br + jnp.where(is0, 0.0, ai * bi)
    pi = jnp.where(is0, ai * bi, ar * bi - ai * br)
    o_ref[0, :, :_F] = pr.T
    o_ref[0, :, _F:] = pi.T


def _pcross_t(af, bf, interpret=False):
    return pl.pallas_call(
        _pcross_t_kernel,
        grid=(_B,),
        in_specs=[
            pl.BlockSpec((1, 2 * _F, _N), lambda b: (b, 0, 0)),
            pl.BlockSpec((1, 2 * _F, _N), lambda b: (b, 0, 0)),
        ],
        out_specs=pl.BlockSpec((1, _N, 2 * _F), lambda b: (b, 0, 0)),
        out_shape=jax.ShapeDtypeStruct((_B, _N, 2 * _F), jnp.float32),
        interpret=interpret,
    )(af, bf)


_R = _B * _N
_NW = 32
_RPW = _R // _NW


def _sc_topk(corrt):
    mesh = plsc.VectorSubcoreMesh(core_axis_name="c", subcore_axis_name="s")
    chunk = 16

    @functools.partial(
        pl.kernel,
        out_type=(jax.ShapeDtypeStruct((_R, 128), jnp.int32),
                  jax.ShapeDtypeStruct((_R, 128), jnp.float32)),
        mesh=mesh,
        scratch_types=[
            pltpu.VMEM((chunk, _L), jnp.float32),
            pltpu.VMEM((_RPW, 128), jnp.int32),
            pltpu.VMEM((_RPW, 128), jnp.float32),
        ],
        compiler_params=pltpu.CompilerParams(needs_layout_passes=False),
    )
    def k(corr_hbm, dly_hbm, wts_hbm, chunk_v, dly_v, wts_v):
        wid = lax.axis_index("s") * 2 + lax.axis_index("c")
        base = wid * _RPW
        lanes = lax.iota(jnp.int32, 16)
        neg = jnp.full((16,), -3.0e38, jnp.float32)
        zero16 = jnp.zeros((16,), jnp.int32)

        def do_chunk(ci, _):
            pltpu.sync_copy(corr_hbm.at[pl.ds(base + ci * chunk, chunk)],
                            chunk_v)

            def topk_pair(pr, _):
                def merge(j, carry):
                    cv0, ci0, cv1, ci1 = carry
                    ix = lanes + j * 16
                    x0 = chunk_v[2 * pr, pl.ds(j * 16, 16)]
                    x1 = chunk_v[2 * pr + 1, pl.ds(j * 16, 16)]
                    s0, si0 = plsc.sort_key_val(x0, ix)
                    s1, si1 = plsc.sort_key_val(x1, ix)
                    m0 = cv0 >= s0
                    m1 = cv1 >= s1
                    cv0, ci0 = plsc.sort_key_val(
                        jnp.where(m0, cv0, s0), jnp.where(m0, ci0, si0),
                        descending=True)
                    cv1, ci1 = plsc.sort_key_val(
                        jnp.where(m1, cv1, s1), jnp.where(m1, ci1, si1),
                        descending=True)
                    return cv0, ci0, cv1, ci1

                cv0, ci0, cv1, ci1 = lax.fori_loop(
                    0, _L // 16, merge, (neg, zero16, neg, zero16))

                def finish(cv, ci, row):
                    e = jnp.exp(cv - jnp.max(cv, axis=0))
                    e = jnp.where(lanes < _TOPK, e, 0.0)
                    w = e / jnp.sum(e, axis=0)
                    dly_v[row, pl.ds(0, 16)] = ci
                    wts_v[row, pl.ds(0, 16)] = w

                finish(cv0, ci0, ci * chunk + 2 * pr)
                finish(cv1, ci1, ci * chunk + 2 * pr + 1)
                return 0

            lax.fori_loop(0, chunk // 2, topk_pair, 0)
            return 0

        lax.fori_loop(0, _RPW // chunk, do_chunk, 0)
        pltpu.sync_copy(dly_v, dly_hbm.at[pl.ds(base, _RPW)])
        pltpu.sync_copy(wts_v, wts_hbm.at[pl.ds(base, _RPW)])

    return k(corrt)


def _wmat_kernel(d_ref, w_ref, o_ref):
    bd = o_ref.shape[1]
    dly = d_ref[0]
    wts = w_ref[0]
    iota_d = (lax.broadcasted_iota(jnp.int32, (bd, _N), 0)
              + pl.program_id(1) * bd)
    acc = jnp.zeros((bd, _N), jnp.float32)
    for i in range(_TOPK):
        acc = acc + jnp.where(iota_d == dly[i:i + 1, :], wts[i:i + 1, :], 0.0)
    o_ref[0] = acc


def _wmat_build(dly_t, wts_t, bd=512, interpret=False):
    return pl.pallas_call(
        _wmat_kernel,
        grid=(_B, _L // bd),
        in_specs=[
            pl.BlockSpec((1, 16, _N), lambda b, i: (b, 0, 0)),
            pl.BlockSpec((1, 16, _N), lambda b, i: (b, 0, 0)),
        ],
        out_specs=pl.BlockSpec((1, bd, _N), lambda b, i: (b, i, 0)),
        out_shape=jax.ShapeDtypeStruct((_B, _L, _N), jnp.float32),
        interpret=interpret,
    )(dly_t, wts_t)


def _topk_kernel(c_ref, o_ref):
    x = c_ref[0]
    iota_d = lax.broadcasted_iota(jnp.int32, (_L, _N), 0)
    neg = jnp.float32(-3.0e38)
    vals = []
    args = []
    for _ in range(_TOPK):
        m = jnp.max(x, axis=0, keepdims=True)
        am = jnp.min(jnp.where(x == m, iota_d, _L), axis=0, keepdims=True)
        x = jnp.where(iota_d == am, neg, x)
        vals.append(m)
        args.append(am)
    V = jnp.concatenate(vals, axis=0)
    e = jnp.exp(V - V[0:1, :])
    sm = e / jnp.sum(e, axis=0, keepdims=True)
    acc = jnp.zeros((_L, _N), jnp.float32)
    for i in range(_TOPK):
        acc = acc + jnp.where(iota_d == args[i], sm[i:i + 1, :], 0.0)
    o_ref[0] = acc


def _topk_weights(corr, interpret=False):
    return pl.pallas_call(
        _topk_kernel,
        grid=(_B,),
        in_specs=[pl.BlockSpec((1, _L, _N), lambda b: (b, 0, 0))],
        out_specs=pl.BlockSpec((1, _L, _N), lambda b: (b, 0, 0)),
        out_shape=jax.ShapeDtypeStruct((_B, _L, _N), jnp.float32),
        interpret=interpret,
    )(corr)


def _run(queries, keys, values, Wq, bq, Wk, bk, Wv, bv, Wo, bo,
         interpret=False):
    cs = jnp.asarray(_CS_NP)[None]
    icst = jnp.asarray(_ICST_NP)[None]
    ics = jnp.asarray(_ICS_NP)[None]
    bo8 = jnp.broadcast_to(bo, (8, _D))

    csh = jnp.asarray(_CS_H)[None]
    csl = jnp.asarray(_CS_L)[None]
    icsth = jnp.asarray(_ICST_H)[None]
    icstl = jnp.asarray(_ICST_L)[None]
    icsh = jnp.asarray(_ICS_H)[None]
    icsl = jnp.asarray(_ICS_L)[None]
    bq8 = jnp.broadcast_to(bq, (8, _N))
    bk8 = jnp.broadcast_to(bk, (8, _N))
    bv8 = jnp.broadcast_to(bv, (8, _N))
    q = _bmm(queries, Wq[None], bq8, interpret=interpret)
    k = _bmm(keys, Wk[None], bk8, interpret=interpret)
    v = _bmm(values, Wv[None], bv8, interpret=interpret)
    qf = _bmm_psl(csh, csl, q, interpret=interpret)
    kf = _bmm_psl(csh, csl, k, interpret=interpret)
    vf = _bmm_psl(csh, csl, v, interpret=interpret)
    corrt = _corrt_fused(qf, kf, icsh, icsl, interpret=interpret)
    dly, wts = _sc_topk(corrt.reshape(_R, _L))
    dly_t = dly[:, :16].reshape(_B, _N, 16).transpose(0, 2, 1)
    wts_t = wts[:, :16].reshape(_B, _N, 16).transpose(0, 2, 1)
    wmat = _wmat_build(dly_t, wts_t, interpret=interpret)
    wf = _bmm_psl(csh, csl, wmat, interpret=interpret)
    p2 = _pcross(vf, wf, interpret=interpret)
    agg = _bmm_psl(icsth, icstl, p2, interpret=interpret)
    return _bmm(agg, Wo[None], bo8, interpret=interpret)


def kernel(queries, keys, values, Wq, bq, Wk, bk, Wv, bv, Wo, bo):
    return _run(queries, keys, values, Wq, bq, Wk, bk, Wv, bv, Wo, bo)
